# baseline (speedup 1.0000x reference)
import functools

import jax
import jax.numpy as jnp
from jax import lax
from jax.experimental import pallas as pl
from jax.experimental.pallas import tpu as pltpu
from jax.experimental.pallas import tpu_sc as plsc

_L, _B, _N, _C = 64, 16, 256, 256
_HROWS = _N // 2
_LANES = 16


def _row_sums(mat, ones_col):
    return lax.dot_general(
        mat, ones_col, (((1,), (0,)), ((), ())),
        preferred_element_type=jnp.float32)


_FB = 64


def _sim_body(x_ref, o_ref, prev_ref, n2p_ref):
    t = pl.program_id(1)

    ones_col = jnp.ones((_C, 128), jnp.float32)
    mean_row = jnp.full((8, _N), 1.0 / (_N * 128), jnp.float32)
    lane = lax.broadcasted_iota(jnp.int32, (1, 1, _L), 2)

    acc = jnp.where(
        t == 0, jnp.full((1, 1, _L), -jnp.inf, jnp.float32), o_ref[...])

    prev = prev_ref[...]
    n2prev = n2p_ref[...]
    for i in range(_FB):
        cur = x_ref[i, 0]
        n2c = jnp.sum(cur * cur, axis=-1, keepdims=True)
        dot = _row_sums(prev * cur, ones_col)
        sim = dot * lax.rsqrt(jnp.maximum(n2prev * n2c, 1e-16))
        avg_mat = lax.dot_general(
            mean_row, sim, (((1,), (0,)), ((), ())),
            preferred_element_type=jnp.float32)
        avg = jnp.sum(avg_mat[0:1, :])
        pair = t * _FB + i - 1
        upd = jnp.where(lane == pair, avg, acc)
        if i == 0:
            acc = jnp.where(t > 0, upd, acc)
        else:
            acc = upd
        prev = cur
        n2prev = n2c

    o_ref[...] = acc
    prev_ref[...] = prev
    n2p_ref[...] = n2prev


def _avg_sims(memory_bank):
    return pl.pallas_call(
        _sim_body,
        grid=(_B, _L // _FB),
        in_specs=[pl.BlockSpec((_FB, 1, _N, _C), lambda b, t: (t, b, 0, 0))],
        out_specs=pl.BlockSpec((1, 1, _L), lambda b, t: (b, 0, 0)),
        out_shape=jax.ShapeDtypeStruct((_B, 1, _L), jnp.float32),
        scratch_shapes=[
            pltpu.VMEM((_N, _C), jnp.float32),
            pltpu.VMEM((_N, 1), jnp.float32),
        ],
    )(memory_bank)


_QROWS = _N // 4


def _ring_copy(bank_hbm, out_hbm, bufs, rsem, wsem, b, j_start, n_frames,
               src_off):
    total = 4 * n_frames

    def src_row(g, q):
        return ((j_start + g + src_off) * _B + b) * _N + q * _QROWS

    def dst_row(g, q):
        return ((j_start + g) * _B + b) * _N + q * _QROWS

    def start_read(g, q, slot):
        pltpu.make_async_copy(
            bank_hbm.at[pl.ds(src_row(g, q), _QROWS)],
            bufs[slot], rsem[slot]).start()

    for m in range(4):
        @pl.when(m < total)
        def _(m=m):
            start_read(m // 4, m % 4, m % 6)

    def block(blk, _):
        u0 = blk * 12
        for m in range(12):
            u = u0 + m
            sl = m % 6
            g = m // 4
            q = m % 4

            @pl.when(u < total)
            def _(u=u, sl=sl, g=g, q=q, blk=blk):
                pltpu.make_async_copy(
                    bank_hbm.at[pl.ds(src_row(blk * 3 + g, q), _QROWS)],
                    bufs[sl], rsem[sl]).wait()
                pltpu.make_async_copy(
                    bufs[sl],
                    out_hbm.at[pl.ds(dst_row(blk * 3 + g, q), _QROWS)],
                    wsem[sl]).start()

            @pl.when(jnp.logical_and(u - 2 >= 0, u - 2 < total))
            def _(u=u, sl=sl, blk=blk):
                ps = (sl - 2) % 6
                pltpu.make_async_copy(
                    bufs[ps], out_hbm.at[pl.ds(dst_row(0, 0), _QROWS)],
                    wsem[ps]).wait()

            @pl.when(u + 4 < total)
            def _(u=u, m=m, blk=blk):
                start_read(blk * 3 + (m + 4) // 4, (m + 4) % 4,
                           (m + 4) % 6)
        return 0

    lax.fori_loop(0, 11, block, 0)


_SC_SCRATCH = (
    [pltpu.VMEM((_QROWS, _C), jnp.float32)] * 6
    + [pltpu.SemaphoreType.DMA] * 12
)


def _worker_ids():
    b = lax.axis_index("s")
    half = lax.axis_index("c")
    j0 = half * 32
    nj = jnp.where(half == 0, 32, _L - 1 - 32)
    return b, j0, nj


def _sc_speccopy(bank_rows):
    mesh = plsc.VectorSubcoreMesh(core_axis_name="c", subcore_axis_name="s")

    @functools.partial(
        pl.kernel,
        mesh=mesh,
        out_type=jax.ShapeDtypeStruct(((_L - 1) * _B * _N, _C), jnp.float32),
        scratch_types=list(_SC_SCRATCH),
    )
    def body(bank_hbm, out_hbm, *ring):
        bufs = list(ring[0:6])
        rsem = list(ring[6:12])
        wsem = list(ring[12:18])
        b, j0, nj = _worker_ids()
        _ring_copy(bank_hbm, out_hbm, bufs, rsem, wsem, b, j0, nj, 0)

    return body(bank_rows)


def _sc_fixup(bank_rows, sims, out0):
    mesh = plsc.VectorSubcoreMesh(core_axis_name="c", subcore_axis_name="s")

    @functools.partial(
        pl.kernel,
        mesh=mesh,
        out_type=(),
        scratch_types=(
            [pltpu.VMEM((_L,), jnp.float32)] + list(_SC_SCRATCH)
        ),
    )
    def body(bank_hbm, sim_hbm, out_hbm, simrow, *ring):
        bufs = list(ring[0:6])
        rsem = list(ring[6:12])
        wsem = list(ring[12:18])
        b, j0, nj = _worker_ids()

        pltpu.sync_copy(sim_hbm.at[pl.ds(b * _L, _L)], simrow)
        best_v = simrow[pl.ds(0, _LANES)]
        best_i = lax.iota(jnp.int32, _LANES)
        for ch in range(1, _L // _LANES):
            v2 = simrow[pl.ds(ch * _LANES, _LANES)]
            i2 = lax.iota(jnp.int32, _LANES) + ch * _LANES
            take2 = v2 > best_v
            best_v = jnp.where(take2, v2, best_v)
            best_i = jnp.where(take2, i2, best_i)
        mx = best_v[0]
        k = best_i[0]
        for lane in range(1, _LANES):
            v = best_v[lane]
            i = best_i[lane]
            better = jnp.logical_or(
                v > mx, jnp.logical_and(v == mx, i < k))
            mx = jnp.where(better, v, mx)
            k = jnp.where(better, i, k)

        jf = jnp.maximum(j0, k)
        njf = jnp.maximum(0, j0 + nj - jf)
        _ring_copy(bank_hbm, out_hbm, bufs, rsem, wsem, b, jf, njf, 1)

        @pl.when(jnp.logical_and(k >= j0, k < j0 + nj))
        def _():
            base_k = (k * _B + b) * _N
            base_k1 = ((k + 1) * _B + b) * _N
            for q in range(4):
                pltpu.sync_copy(
                    bank_hbm.at[pl.ds(base_k + q * _QROWS, _QROWS)], bufs[0])
                pltpu.sync_copy(
                    bank_hbm.at[pl.ds(base_k1 + q * _QROWS, _QROWS)],
                    bufs[1])

                def avg_row(r, _):
                    def avg_chunk(cc, _):
                        sl = pl.ds(cc * _LANES, _LANES)
                        bufs[0][r, sl] = (
                            bufs[0][r, sl] + bufs[1][r, sl]) * 0.5
                        return 0

                    lax.fori_loop(0, _C // _LANES, avg_chunk, 0)
                    return 0

                lax.fori_loop(0, _QROWS, avg_row, 0)
                pltpu.sync_copy(
                    bufs[0], out_hbm.at[pl.ds(base_k + q * _QROWS, _QROWS)])

    out_ref = jax.new_ref(out0)
    body(bank_rows, sims, out_ref)
    return jax.freeze(out_ref)


def kernel(memory_bank):
    L, B, N, C = memory_bank.shape
    bank_rows = memory_bank.reshape(L * B * N, C)
    out0 = _sc_speccopy(bank_rows)
    sims = _avg_sims(memory_bank).reshape(B * L)
    out_rows = _sc_fixup(bank_rows, sims, out0)
    return out_rows.reshape(L - 1, B, N, C)


# speedup vs baseline: 1.0289x; 1.0289x over previous
import functools

import jax
import jax.numpy as jnp
from jax import lax
from jax.experimental import pallas as pl
from jax.experimental.pallas import tpu as pltpu
from jax.experimental.pallas import tpu_sc as plsc

_L, _B, _N, _C = 64, 16, 256, 256
_HROWS = _N // 2
_LANES = 16


def _row_sums(mat, ones_col):
    return lax.dot_general(
        mat, ones_col, (((1,), (0,)), ((), ())),
        preferred_element_type=jnp.float32)


_FB = 64


def _sim_body(x_ref, o_ref, prev_ref, n2p_ref):
    t = pl.program_id(1)

    ones_col = jnp.ones((_C, 128), jnp.float32)
    mean_row = jnp.full((8, _N), 1.0 / (_N * 128), jnp.float32)
    lane = lax.broadcasted_iota(jnp.int32, (1, 1, _L), 2)

    acc = jnp.where(
        t == 0, jnp.full((1, 1, _L), -jnp.inf, jnp.float32), o_ref[...])

    prev = prev_ref[...]
    n2prev = n2p_ref[...]
    for i in range(_FB):
        cur = x_ref[i, 0]
        n2c = jnp.sum(cur * cur, axis=-1, keepdims=True)
        dot = _row_sums(prev * cur, ones_col)
        sim = dot * lax.rsqrt(jnp.maximum(n2prev * n2c, 1e-16))
        avg_mat = lax.dot_general(
            mean_row, sim, (((1,), (0,)), ((), ())),
            preferred_element_type=jnp.float32)
        avg = jnp.sum(avg_mat[0:1, :])
        pair = t * _FB + i - 1
        upd = jnp.where(lane == pair, avg, acc)
        if i == 0:
            acc = jnp.where(t > 0, upd, acc)
        else:
            acc = upd
        prev = cur
        n2prev = n2c

    o_ref[...] = acc
    prev_ref[...] = prev
    n2p_ref[...] = n2prev


def _avg_sims(memory_bank):
    return pl.pallas_call(
        _sim_body,
        grid=(_B, _L // _FB),
        in_specs=[pl.BlockSpec((_FB, 1, _N, _C), lambda b, t: (t, b, 0, 0))],
        out_specs=pl.BlockSpec((1, 1, _L), lambda b, t: (b, 0, 0)),
        out_shape=jax.ShapeDtypeStruct((_B, 1, _L), jnp.float32),
        scratch_shapes=[
            pltpu.VMEM((_N, _C), jnp.float32),
            pltpu.VMEM((_N, 1), jnp.float32),
        ],
    )(memory_bank)


_QROWS = _N // 4


def _ring_copy(bank_hbm, out_hbm, bufs, rsem, wsem, b, j_start, n_frames,
               src_off):
    total = 4 * n_frames

    def src_row(g, q):
        return ((j_start + g + src_off) * _B + b) * _N + q * _QROWS

    def dst_row(g, q):
        return ((j_start + g) * _B + b) * _N + q * _QROWS

    def start_read(g, q, slot):
        pltpu.make_async_copy(
            bank_hbm.at[pl.ds(src_row(g, q), _QROWS)],
            bufs[slot], rsem[slot]).start()

    for m in range(4):
        @pl.when(m < total)
        def _(m=m):
            start_read(m // 4, m % 4, m % 6)

    def block(blk, _):
        u0 = blk * 12
        for m in range(12):
            u = u0 + m
            sl = m % 6
            g = m // 4
            q = m % 4

            @pl.when(u < total)
            def _(u=u, sl=sl, g=g, q=q, blk=blk):
                pltpu.make_async_copy(
                    bank_hbm.at[pl.ds(src_row(blk * 3 + g, q), _QROWS)],
                    bufs[sl], rsem[sl]).wait()
                pltpu.make_async_copy(
                    bufs[sl],
                    out_hbm.at[pl.ds(dst_row(blk * 3 + g, q), _QROWS)],
                    wsem[sl]).start()

            @pl.when(jnp.logical_and(u - 2 >= 0, u - 2 < total))
            def _(u=u, sl=sl, blk=blk):
                ps = (sl - 2) % 6
                pltpu.make_async_copy(
                    bufs[ps], out_hbm.at[pl.ds(dst_row(0, 0), _QROWS)],
                    wsem[ps]).wait()

            @pl.when(u + 4 < total)
            def _(u=u, m=m, blk=blk):
                start_read(blk * 3 + (m + 4) // 4, (m + 4) % 4,
                           (m + 4) % 6)
        return 0

    lax.fori_loop(0, 11, block, 0)


_SC_SCRATCH = (
    [pltpu.VMEM((_QROWS, _C), jnp.float32)] * 6
    + [pltpu.SemaphoreType.DMA] * 12
)


_JSPEC = 40


def _sc_speccopy(bank_rows):
    mesh = plsc.VectorSubcoreMesh(core_axis_name="c", subcore_axis_name="s")

    @functools.partial(
        pl.kernel,
        mesh=mesh,
        out_type=jax.ShapeDtypeStruct(((_L - 1) * _B * _N, _C), jnp.float32),
        scratch_types=list(_SC_SCRATCH),
    )
    def body(bank_hbm, out_hbm, *ring):
        bufs = list(ring[0:6])
        rsem = list(ring[6:12])
        wsem = list(ring[12:18])
        b = lax.axis_index("s")
        half = lax.axis_index("c")
        _ring_copy(bank_hbm, out_hbm, bufs, rsem, wsem, b,
                   half * (_JSPEC // 2), _JSPEC // 2, 0)

    return body(bank_rows)


def _sc_fixup(bank_rows, sims, out0):
    mesh = plsc.VectorSubcoreMesh(core_axis_name="c", subcore_axis_name="s")

    @functools.partial(
        pl.kernel,
        mesh=mesh,
        out_type=(),
        scratch_types=(
            [pltpu.VMEM((_L,), jnp.float32)] + list(_SC_SCRATCH)
        ),
    )
    def body(bank_hbm, sim_hbm, out_hbm, simrow, *ring):
        bufs = list(ring[0:6])
        rsem = list(ring[6:12])
        wsem = list(ring[12:18])
        b = lax.axis_index("s")
        half = lax.axis_index("c")

        pltpu.sync_copy(sim_hbm.at[pl.ds(b * _L, _L)], simrow)
        best_v = simrow[pl.ds(0, _LANES)]
        best_i = lax.iota(jnp.int32, _LANES)
        for ch in range(1, _L // _LANES):
            v2 = simrow[pl.ds(ch * _LANES, _LANES)]
            i2 = lax.iota(jnp.int32, _LANES) + ch * _LANES
            take2 = v2 > best_v
            best_v = jnp.where(take2, v2, best_v)
            best_i = jnp.where(take2, i2, best_i)
        mx = best_v[0]
        k = best_i[0]
        for lane in range(1, _LANES):
            v = best_v[lane]
            i = best_i[lane]
            better = jnp.logical_or(
                v > mx, jnp.logical_and(v == mx, i < k))
            mx = jnp.where(better, v, mx)
            k = jnp.where(better, i, k)

        n_a = jnp.maximum(0, k - _JSPEC)
        n_a0 = lax.shift_right_logical(n_a + 1, 1)
        a_start = jnp.where(half == 0, _JSPEC, _JSPEC + n_a0)
        a_count = jnp.where(half == 0, n_a0, n_a - n_a0)
        _ring_copy(bank_hbm, out_hbm, bufs, rsem, wsem, b,
                   a_start, a_count, 0)

        n_b = (_L - 1) - k
        n_b0 = lax.shift_right_logical(n_b + 1, 1)
        b_start = jnp.where(half == 0, k, k + n_b0)
        b_count = jnp.where(half == 0, n_b0, n_b - n_b0)
        _ring_copy(bank_hbm, out_hbm, bufs, rsem, wsem, b,
                   b_start, b_count, 1)

        @pl.when(half == 0)
        def _():
            base_k = (k * _B + b) * _N
            base_k1 = ((k + 1) * _B + b) * _N
            for q in range(4):
                pltpu.sync_copy(
                    bank_hbm.at[pl.ds(base_k + q * _QROWS, _QROWS)], bufs[0])
                pltpu.sync_copy(
                    bank_hbm.at[pl.ds(base_k1 + q * _QROWS, _QROWS)],
                    bufs[1])

                def avg_row(r, _):
                    def avg_chunk(cc, _):
                        sl = pl.ds(cc * _LANES, _LANES)
                        bufs[0][r, sl] = (
                            bufs[0][r, sl] + bufs[1][r, sl]) * 0.5
                        return 0

                    lax.fori_loop(0, _C // _LANES, avg_chunk, 0)
                    return 0

                lax.fori_loop(0, _QROWS, avg_row, 0)
                pltpu.sync_copy(
                    bufs[0], out_hbm.at[pl.ds(base_k + q * _QROWS, _QROWS)])

    out_ref = jax.new_ref(out0)
    body(bank_rows, sims, out_ref)
    return jax.freeze(out_ref)


def kernel(memory_bank):
    L, B, N, C = memory_bank.shape
    bank_rows = memory_bank.reshape(L * B * N, C)
    out0 = _sc_speccopy(bank_rows)
    sims = _avg_sims(memory_bank).reshape(B * L)
    out_rows = _sc_fixup(bank_rows, sims, out0)
    return out_rows.reshape(L - 1, B, N, C)
